# baseline (speedup 1.0000x reference)
import functools

import jax
import jax.numpy as jnp
from jax import lax
from jax.experimental import pallas as pl
from jax.experimental.pallas import tpu as pltpu
from jax.experimental.pallas import tpu_sc as plsc

B, S = 4096, 200
D = 64
TOTAL = B * S
IW = 128
CHUNK_ROWS = 8
C = CHUNK_ROWS * IW


def _make_sc_kernel():
    info = plsc.get_sparse_core_info()
    nc, ns = info.num_cores, info.num_subcores
    nw = nc * ns
    per_w = TOTAL // nw
    n_chunks = per_w // C
    idx_rows_per_w = per_w // IW

    mesh = plsc.VectorSubcoreMesh(core_axis_name="c", subcore_axis_name="s")

    @functools.partial(
        pl.kernel,
        mesh=mesh,
        out_type=jax.ShapeDtypeStruct((TOTAL, D), jnp.float32),
        scratch_types=[
            pltpu.VMEM((CHUNK_ROWS, IW), jnp.int32),
            pltpu.VMEM((C, D), jnp.float32),
            pltpu.SemaphoreType.DMA,
        ],
        compiler_params=pltpu.CompilerParams(use_tc_tiling_on_sc=False),
    )
    def emb(table_hbm, idx_hbm, out_hbm, idx_v, rows_v, sem):
        wid = lax.axis_index("s") * nc + lax.axis_index("c")
        row0 = wid * idx_rows_per_w
        base = wid * per_w

        def body(i, carry):
            pltpu.sync_copy(
                idx_hbm.at[pl.ds(row0 + i * CHUNK_ROWS, CHUNK_ROWS)], idx_v)
            copies = []
            for j in range(CHUNK_ROWS):
                copies.append(pltpu.async_copy(
                    table_hbm.at[idx_v.at[j]],
                    rows_v.at[pl.ds(j * IW, IW)],
                    sem))
            for cp in copies:
                cp.wait()
            pltpu.sync_copy(rows_v, out_hbm.at[pl.ds(base + i * C, C)])
            return carry

        lax.fori_loop(0, n_chunks, body, 0)

    return emb


_sc_emb = _make_sc_kernel()


def kernel(x, weight):
    idx = x.reshape(TOTAL // IW, IW).astype(jnp.int32)
    out = _sc_emb(weight, idx)
    return out.reshape(B, S, D)


# speedup vs baseline: 1.0132x; 1.0132x over previous
import functools

import jax
import jax.numpy as jnp
from jax import lax
from jax.experimental import pallas as pl
from jax.experimental.pallas import tpu as pltpu
from jax.experimental.pallas import tpu_sc as plsc

B, S = 4096, 200
D = 64
TOTAL = B * S
IW = 128
CHUNK_ROWS = 4
C = CHUNK_ROWS * IW


def _make_sc_kernel():
    info = plsc.get_sparse_core_info()
    nc, ns = info.num_cores, info.num_subcores
    nw = nc * ns
    per_w = TOTAL // nw
    n_chunks = per_w // C
    idx_rows_per_w = per_w // IW
    n_it = n_chunks // 2

    mesh = plsc.VectorSubcoreMesh(core_axis_name="c", subcore_axis_name="s")

    @functools.partial(
        pl.kernel,
        mesh=mesh,
        out_type=jax.ShapeDtypeStruct((TOTAL, D), jnp.float32),
        scratch_types=[
            pltpu.VMEM((idx_rows_per_w, IW), jnp.int32),
            pltpu.VMEM((C, D), jnp.float32),
            pltpu.VMEM((C, D), jnp.float32),
            pltpu.SemaphoreType.DMA,
            pltpu.SemaphoreType.DMA,
            pltpu.SemaphoreType.DMA,
            pltpu.SemaphoreType.DMA,
        ],
        compiler_params=pltpu.CompilerParams(use_tc_tiling_on_sc=False),
    )
    def emb(table_hbm, idx_hbm, out_hbm, idx_v, rows0, rows1,
            gsem0, gsem1, ssem0, ssem1):
        wid = lax.axis_index("s") * nc + lax.axis_index("c")
        row0 = wid * idx_rows_per_w
        base = wid * per_w
        rows = (rows0, rows1)
        gsem = (gsem0, gsem1)
        ssem = (ssem0, ssem1)

        pltpu.sync_copy(idx_hbm.at[pl.ds(row0, idx_rows_per_w)], idx_v)

        def fire_gathers(ch, p):
            for j in range(CHUNK_ROWS):
                pltpu.async_copy(
                    table_hbm.at[idx_v.at[ch * CHUNK_ROWS + j]],
                    rows[p].at[pl.ds(j * IW, IW)],
                    gsem[p])

        def drain_gathers(p):
            for j in range(CHUNK_ROWS):
                pltpu.make_async_copy(
                    table_hbm.at[pl.ds(0, IW)],
                    rows[p].at[pl.ds(j * IW, IW)],
                    gsem[p]).wait()

        def fire_store(ch, p):
            pltpu.async_copy(rows[p], out_hbm.at[pl.ds(base + ch * C, C)],
                             ssem[p])

        def wait_store(p):
            pltpu.make_async_copy(rows[p], out_hbm.at[pl.ds(base, C)],
                                  ssem[p]).wait()

        fire_gathers(0, 0)

        def body(it, carry):
            ch0 = it * 2
            drain_gathers(0)
            fire_store(ch0, 0)

            @pl.when(it > 0)
            def _():
                wait_store(1)
            fire_gathers(ch0 + 1, 1)

            drain_gathers(1)
            fire_store(ch0 + 1, 1)
            wait_store(0)

            @pl.when(it < n_it - 1)
            def _():
                fire_gathers(ch0 + 2, 0)
            return carry

        lax.fori_loop(0, n_it, body, 0)
        wait_store(1)

    return emb


_sc_emb = _make_sc_kernel()


def kernel(x, weight):
    idx = x.reshape(TOTAL // IW, IW).astype(jnp.int32)
    out = _sc_emb(weight, idx)
    return out.reshape(B, S, D)
